# baseline (speedup 1.0000x reference)
import functools

import jax
import jax.numpy as jnp
from jax import lax
from jax.experimental import pallas as pl
from jax.experimental.pallas import tpu as pltpu
from jax.experimental.pallas import tpu_sc as plsc

_B = 4096
_L = 200
_D = 64


def _make_sc_gather():
    info = plsc.get_sparse_core_info()
    nw = info.num_cores * info.num_subcores
    b_per_w = _B // nw
    mesh = plsc.VectorSubcoreMesh(core_axis_name="c", subcore_axis_name="s")

    @functools.partial(
        pl.kernel,
        mesh=mesh,
        out_type=jax.ShapeDtypeStruct((_B, 2 * _D), jnp.float32),
        scratch_types=[
            pltpu.VMEM((b_per_w,), jnp.int32),
            pltpu.VMEM((b_per_w,), jnp.int32),
            pltpu.VMEM((b_per_w, 2 * _D), jnp.float32),
            pltpu.SemaphoreType.DMA,
        ],
    )
    def sc_gather(table2_hbm, idx_hbm, out_hbm, idx_v, pair_v, rows_v, sem):
        wid = lax.axis_index("s") * info.num_cores + lax.axis_index("c")
        base = wid * b_per_w
        pltpu.sync_copy(idx_hbm.at[pl.ds(base, b_per_w)], idx_v)
        for j in range(b_per_w // 16):
            pair_v[pl.ds(j * 16, 16)] = idx_v[pl.ds(j * 16, 16)] >> 1
        pltpu.async_copy(table2_hbm.at[pair_v], rows_v, sem).wait()
        pltpu.sync_copy(rows_v, out_hbm.at[pl.ds(base, b_per_w)])

    return sc_gather


_sc_gather = _make_sc_gather()

_BB = 128


def _copy_body(feat_ref, out_ref):
    out_ref[1:, :, :] = feat_ref[...]


_copy_feat = pl.pallas_call(
    _copy_body,
    grid=(_B // _BB,),
    in_specs=[pl.BlockSpec((_L, _D, _BB), lambda i: (0, 0, i))],
    out_specs=pl.BlockSpec((_L + 1, _D, _BB), lambda i: (0, 0, i)),
    out_shape=jax.ShapeDtypeStruct((_L + 1, _D, _B), jnp.float32),
)


def _patch_body(emb_ref, prev_ref, out_ref):
    del prev_ref
    out_ref[...] = emb_ref[...]


_patch = pl.pallas_call(
    _patch_body,
    grid=(1,),
    in_specs=[
        pl.BlockSpec((1, _D, _B), lambda i: (0, 0, 0)),
        pl.BlockSpec(memory_space=pl.ANY),
    ],
    out_specs=pl.BlockSpec((1, _D, _B), lambda i: (0, 0, 0)),
    out_shape=jax.ShapeDtypeStruct((_L + 1, _D, _B), jnp.float32),
    input_output_aliases={1: 0},
)


def kernel(feature, index, table):
    idx = index.reshape(-1).astype(jnp.int32)
    feat_t = jnp.transpose(feature, (1, 2, 0))
    table2 = table.reshape(500000, 2 * _D)
    pairs = _sc_gather(table2, idx)
    emb = jnp.where((idx & 1)[:, None] == 1, pairs[:, _D:], pairs[:, :_D])
    emb_t = jnp.transpose(emb)[None]
    out_t = _copy_feat(feat_t)
    out_t = _patch(emb_t, out_t)
    return jnp.transpose(out_t, (2, 0, 1))


# speedup vs baseline: 2.0439x; 2.0439x over previous
import functools

import jax
import jax.numpy as jnp
from jax import lax
from jax.experimental import pallas as pl
from jax.experimental.pallas import tpu as pltpu
from jax.experimental.pallas import tpu_sc as plsc

_B = 4096
_L = 200
_D = 64


def _make_sc_gather():
    info = plsc.get_sparse_core_info()
    nw = info.num_cores * info.num_subcores
    b_per_w = _B // nw
    mesh = plsc.VectorSubcoreMesh(core_axis_name="c", subcore_axis_name="s")

    n_rounds = 2
    s_per_round = b_per_w // n_rounds

    @functools.partial(
        pl.kernel,
        mesh=mesh,
        out_type=jax.ShapeDtypeStruct((_D, _B), jnp.float32),
        scratch_types=[
            pltpu.VMEM((b_per_w,), jnp.int32),
            pltpu.VMEM((s_per_round, 8, _D), jnp.float32),
            pltpu.VMEM((_D, b_per_w), jnp.float32),
            pltpu.SemaphoreType.DMA,
        ],
        compiler_params=pltpu.CompilerParams(needs_layout_passes=False),
    )
    def sc_gather(table3_hbm, idx_hbm, out_hbm, idx_v, rows_v, embt_v, sem):
        wid = lax.axis_index("s") * info.num_cores + lax.axis_index("c")
        base = wid * b_per_w
        pltpu.sync_copy(idx_hbm.at[pl.ds(base, b_per_w)], idx_v)
        iota16 = lax.iota(jnp.int32, 16)
        for rnd in range(n_rounds):
            copies = []
            for j in range(s_per_round):
                off = rnd * s_per_round + j
                chunk = idx_v[pl.ds((off // 16) * 16, 16)]
                slab_s = jnp.sum(
                    jnp.where(iota16 == (off % 16), chunk >> 3, 0), axis=0
                )
                c = pltpu.make_async_copy(
                    table3_hbm.at[pl.ds(slab_s, 1)],
                    rows_v.at[pl.ds(j, 1)],
                    sem,
                )
                c.start()
                copies.append(c)
            for c in copies:
                c.wait()
            for g in range(s_per_round // 16):
                off = rnd * s_per_round + g * 16
                rows16 = idx_v[pl.ds(off, 16)] & 7
                slabs16 = iota16 + g * 16
                for d in range(_D):
                    cols16 = jnp.full((16,), d, jnp.int32)
                    vals = plsc.load_gather(rows_v, [slabs16, rows16, cols16])
                    embt_v[d, pl.ds(off, 16)] = vals
        pltpu.sync_copy(embt_v, out_hbm.at[:, pl.ds(base, b_per_w)])

    return sc_gather


_sc_gather = _make_sc_gather()

_BB = 128


def _copy_body(feat_ref, out_ref):
    out_ref[1:, :, :] = feat_ref[...]


_copy_feat = pl.pallas_call(
    _copy_body,
    grid=(_B // _BB,),
    in_specs=[pl.BlockSpec((_L, _D, _BB), lambda i: (0, 0, i))],
    out_specs=pl.BlockSpec((_L + 1, _D, _BB), lambda i: (0, 0, i)),
    out_shape=jax.ShapeDtypeStruct((_L + 1, _D, _B), jnp.float32),
)


def _patch_body(emb_ref, prev_ref, out_ref):
    del prev_ref
    out_ref[...] = emb_ref[...]


_patch = pl.pallas_call(
    _patch_body,
    grid=(1,),
    in_specs=[
        pl.BlockSpec((1, _D, _B), lambda i: (0, 0, 0)),
        pl.BlockSpec(memory_space=pl.ANY),
    ],
    out_specs=pl.BlockSpec((1, _D, _B), lambda i: (0, 0, 0)),
    out_shape=jax.ShapeDtypeStruct((_L + 1, _D, _B), jnp.float32),
    input_output_aliases={1: 0},
)


def kernel(feature, index, table):
    idx = index.reshape(-1).astype(jnp.int32)
    feat_t = jnp.transpose(feature, (1, 2, 0))
    table3 = table.reshape(125000, 8, _D)
    emb_t = _sc_gather(table3, idx)[None]
    out_t = _copy_feat(feat_t)
    out_t = _patch(emb_t, out_t)
    return jnp.transpose(out_t, (2, 0, 1))
